# Initial kernel scaffold; baseline (speedup 1.0000x reference)
import jax
import jax.numpy as jnp
from jax.experimental import pallas as pl


def kernel(x, edge_attr, pe, edge_index, enc_W1, enc_b1, enc_W2, enc_b2, ee_W1, ee_b1, ee_W2, ee_b2, eu_W1, eu_b1, eu_W2, eu_b2, nu_W1, nu_b1, nu_W2, nu_b2, fu_W1, fu_b1, fu_W2, fu_b2, dec_W1, dec_b1, dec_W2, dec_b2):
    raise NotImplementedError("write your pallas kernel here")


# speedup vs baseline: 1.8427x; 1.8427x over previous
import functools

import jax
import jax.numpy as jnp
from jax import lax
from jax.experimental import pallas as pl
from jax.experimental.pallas import tpu as pltpu
from jax.experimental.pallas import tpu_sc as plsc

N = 10000
E = 320000
D_IN = 128
PE = 16
H = 64
OUT = 128
T = 3
L = 3

RN = 2000
RE = 4000

_NC = 2
_NS = 16
_NW = _NC * _NS
_EPW = E // _NW
_CH = 80
_NCHUNK = _EPW // _CH
_NPS = N // _NS

@functools.cache
def _sc_mesh():
    return plsc.VectorSubcoreMesh(core_axis_name="c", subcore_axis_name="s",
                                  num_cores=_NC, num_subcores=_NS)


def _relu(v):
    return jnp.maximum(v, 0.0)


def _dot(a, b):
    return jnp.dot(a, b, preferred_element_type=jnp.float32)


def _full(a):
    return pl.BlockSpec(a.shape, lambda i: (0,) * a.ndim)


def _rows(shape):
    nd = len(shape)
    return pl.BlockSpec(shape, lambda i: (i,) + (0,) * (nd - 1))


def _enc_body(x_ref, pe_ref, w1x, w1p, b1, w2, b2, wa, wb,
              h_ref, a_ref, b_ref):
    xb = x_ref[...]
    peb = pe_ref[...]
    nt = jnp.clip(jnp.abs(xb[:, 2:3]).astype(jnp.int32), 0, T - 1)
    h = jnp.zeros((RN, H), jnp.float32)
    for t in range(T):
        h1 = _relu(_dot(xb, w1x[t]) + _dot(peb, w1p[t]) + b1[t])
        ht = _dot(h1, w2[t]) + b2[t]
        h = jnp.where(nt == t, ht, h)
    h_ref[...] = h
    a_ref[...] = _dot(h, wa[...])
    b_ref[...] = _dot(h, wb[...])


def _enc_call(x, pe, w1x, w1p, b1, w2, b2, wa, wb):
    return pl.pallas_call(
        _enc_body,
        grid=(N // RN,),
        in_specs=[_rows((RN, D_IN)), _rows((RN, PE))]
        + [_full(w) for w in (w1x, w1p, b1, w2, b2, wa, wb)],
        out_specs=[_rows((RN, H))] * 3,
        out_shape=[jax.ShapeDtypeStruct((N, H), jnp.float32)] * 3,
    )(x, pe, w1x, w1p, b1, w2, b2, wa, wb)


def _eenc_body(ea_ref, w1, b1, w2, b2, wc, bc, he_ref, c_ref):
    h1 = _relu(_dot(ea_ref[...], w1[...]) + b1[...])
    he = _dot(h1, w2[...]) + b2[...]
    he_ref[...] = he
    c_ref[...] = _dot(he, wc[...]) + bc[...]


def _eenc_call(ea, w1, b1, w2, b2, wc, bc):
    return pl.pallas_call(
        _eenc_body,
        grid=(E // RE,),
        in_specs=[_rows((RE, PE))] + [_full(w) for w in (w1, b1, w2, b2, wc, bc)],
        out_specs=[_rows((RE, H))] * 2,
        out_shape=[jax.ShapeDtypeStruct((E, H), jnp.float32)] * 2,
    )(ea, w1, b1, w2, b2, wc, bc)


def _eupd_body(g_ref, c_ref, he_ref, w2, b2, wcn, bcn, out_he, out_c):
    z = _relu(g_ref[...] + c_ref[...])
    hen = he_ref[...] + _dot(z, w2[...]) + b2[...]
    out_he[...] = hen
    out_c[...] = _dot(hen, wcn[...]) + bcn[...]


def _eupd_last_body(g_ref, c_ref, he_ref, w2, b2, out_he):
    z = _relu(g_ref[...] + c_ref[...])
    out_he[...] = he_ref[...] + _dot(z, w2[...]) + b2[...]


def _eupd_call(g, c, he, w2, b2, wcn=None, bcn=None):
    if wcn is None:
        return pl.pallas_call(
            _eupd_last_body,
            grid=(E // RE,),
            in_specs=[_rows((RE, H))] * 3 + [_full(w) for w in (w2, b2)],
            out_specs=[_rows((RE, H))],
            out_shape=[jax.ShapeDtypeStruct((E, H), jnp.float32)],
        )(g, c, he, w2, b2)[0]
    return pl.pallas_call(
        _eupd_body,
        grid=(E // RE,),
        in_specs=[_rows((RE, H))] * 3 + [_full(w) for w in (w2, b2, wcn, bcn)],
        out_specs=[_rows((RE, H))] * 2,
        out_shape=[jax.ShapeDtypeStruct((E, H), jnp.float32)] * 2,
    )(g, c, he, w2, b2, wcn, bcn)


def _nupd_body(h_ref, m0_ref, m1_ref, w1h, w1m, b1, w2, b2,
               f1, fb1, f2, fb2, wa, wb, h_out, a_out, b_out):
    h = h_ref[...]
    m = m0_ref[...] + m1_ref[...]
    u = _dot(_relu(_dot(h, w1h[...]) + _dot(m, w1m[...]) + b1[...]), w2[...]) + b2[...]
    local = h + u
    hn = _dot(_relu(_dot(local, f1[...]) + fb1[...]), f2[...]) + fb2[...]
    h_out[...] = hn
    a_out[...] = _dot(hn, wa[...])
    b_out[...] = _dot(hn, wb[...])


def _nupd_last_body(h_ref, m0_ref, m1_ref, w1h, w1m, b1, w2, b2,
                    f1, fb1, f2, fb2, h_out):
    h = h_ref[...]
    m = m0_ref[...] + m1_ref[...]
    u = _dot(_relu(_dot(h, w1h[...]) + _dot(m, w1m[...]) + b1[...]), w2[...]) + b2[...]
    local = h + u
    h_out[...] = _dot(_relu(_dot(local, f1[...]) + fb1[...]), f2[...]) + fb2[...]


def _nupd_call(h, m0, m1, w1h, w1m, b1, w2, b2, f1, fb1, f2, fb2,
               wa=None, wb=None):
    if wa is None:
        return pl.pallas_call(
            _nupd_last_body,
            grid=(N // RN,),
            in_specs=[_rows((RN, H))] * 3
            + [_full(w) for w in (w1h, w1m, b1, w2, b2, f1, fb1, f2, fb2)],
            out_specs=[_rows((RN, H))],
            out_shape=[jax.ShapeDtypeStruct((N, H), jnp.float32)],
        )(h, m0, m1, w1h, w1m, b1, w2, b2, f1, fb1, f2, fb2)[0]
    return pl.pallas_call(
        _nupd_body,
        grid=(N // RN,),
        in_specs=[_rows((RN, H))] * 3
        + [_full(w) for w in (w1h, w1m, b1, w2, b2, f1, fb1, f2, fb2, wa, wb)],
        out_specs=[_rows((RN, H))] * 3,
        out_shape=[jax.ShapeDtypeStruct((N, H), jnp.float32)] * 3,
    )(h, m0, m1, w1h, w1m, b1, w2, b2, f1, fb1, f2, fb2, wa, wb)


def _dec_body(x_ref, h_ref, w1, b1, w2, b2, out_ref):
    nt = jnp.clip(jnp.abs(x_ref[:, 2:3]).astype(jnp.int32), 0, T - 1)
    h = h_ref[...]
    out = jnp.zeros((RN, OUT), jnp.float32)
    for t in range(T):
        ot = _dot(_relu(_dot(h, w1[t]) + b1[t]), w2[t]) + b2[t]
        out = jnp.where(nt == t, ot, out)
    out_ref[...] = out


def _dec_call(x, h, w1, b1, w2, b2):
    return pl.pallas_call(
        _dec_body,
        grid=(N // RN,),
        in_specs=[_rows((RN, D_IN)), _rows((RN, H))]
        + [_full(w) for w in (w1, b1, w2, b2)],
        out_specs=_rows((RN, OUT)),
        out_shape=jax.ShapeDtypeStruct((N, OUT), jnp.float32),
    )(x, h, w1, b1, w2, b2)


@functools.cache
def _sc_gather_add_kernel():
    return pl.kernel(
        _sc_gather_add_body,
        out_type=jax.ShapeDtypeStruct((E, H), jnp.float32),
        mesh=_sc_mesh(),
        scratch_types=[
            pltpu.VMEM((_CH,), jnp.int32),
            pltpu.VMEM((_CH,), jnp.int32),
            pltpu.VMEM((_CH, H), jnp.float32),
            pltpu.VMEM((_CH, H), jnp.float32),
        ],
        compiler_params=pltpu.CompilerParams(use_tc_tiling_on_sc=False),
    )


def _sc_gather_add(a, b, src, dst):
    return _sc_gather_add_kernel()(a, b, src, dst)


def _sc_gather_add_body(a_hbm, b_hbm, src_hbm, dst_hbm, out_hbm, si, di, ra, rb):
    wid = lax.axis_index("s") * _NC + lax.axis_index("c")
    base = wid * _EPW

    def chunk(j, carry):
        off = base + j * _CH
        pltpu.sync_copy(src_hbm.at[pl.ds(off, _CH)], si)
        pltpu.sync_copy(dst_hbm.at[pl.ds(off, _CH)], di)
        pltpu.sync_copy(a_hbm.at[si], ra)
        pltpu.sync_copy(b_hbm.at[di], rb)

        def row(r, c2):
            for cc in range(H // 16):
                sl = pl.ds(cc * 16, 16)
                ra[r, sl] = ra[r, sl] + rb[r, sl]
            return c2

        lax.fori_loop(0, _CH, row, 0)
        pltpu.sync_copy(ra, out_hbm.at[pl.ds(off, _CH)])
        return carry

    lax.fori_loop(0, _NCHUNK, chunk, 0)


@functools.cache
def _sc_scatter_add_kernel():
    return pl.kernel(
        _sc_scatter_add_body,
        out_type=jax.ShapeDtypeStruct((_NC, N, H), jnp.float32),
        mesh=_sc_mesh(),
        scratch_types=[
            pltpu.VMEM((_CH,), jnp.int32),
            pltpu.VMEM((_CH, H), jnp.float32),
            pltpu.VMEM((_NPS, H), jnp.float32),
            pltpu.VMEM_SHARED((N, H), jnp.float32),
        ],
        compiler_params=pltpu.CompilerParams(use_tc_tiling_on_sc=False),
    )


def _sc_scatter_add(he, dst):
    return _sc_scatter_add_kernel()(he, dst)


def _sc_scatter_add_body(he_hbm, dst_hbm, out_hbm, di, rows, zbuf, shared):
    cid = lax.axis_index("c")
    sid = lax.axis_index("s")
    wid = sid * _NC + cid

    def zrow(r, c2):
        for cc in range(H // 16):
            zbuf[r, pl.ds(cc * 16, 16)] = jnp.zeros((16,), jnp.float32)
        return c2

    lax.fori_loop(0, _NPS, zrow, 0)
    pltpu.sync_copy(zbuf, shared.at[pl.ds(sid * _NPS, _NPS)])
    plsc.subcore_barrier()

    base = wid * _EPW

    def chunk(j, carry):
        off = base + j * _CH
        pltpu.sync_copy(dst_hbm.at[pl.ds(off, _CH)], di)
        pltpu.sync_copy(he_hbm.at[pl.ds(off, _CH)], rows)
        pltpu.sync_copy(rows, shared.at[di], add=True)
        return carry

    lax.fori_loop(0, _NCHUNK, chunk, 0)
    plsc.subcore_barrier()
    pltpu.sync_copy(shared.at[pl.ds(sid * _NPS, _NPS)],
                    out_hbm.at[cid, pl.ds(sid * _NPS, _NPS)])


def kernel(x, edge_attr, pe, edge_index,
           enc_W1, enc_b1, enc_W2, enc_b2,
           ee_W1, ee_b1, ee_W2, ee_b2,
           eu_W1, eu_b1, eu_W2, eu_b2,
           nu_W1, nu_b1, nu_W2, nu_b2,
           fu_W1, fu_b1, fu_W2, fu_b2,
           dec_W1, dec_b1, dec_W2, dec_b2):
    src = edge_index[0].astype(jnp.int32)
    dst = edge_index[1].astype(jnp.int32)

    enc_w1x = enc_W1[:, :D_IN, :]
    enc_w1p = enc_W1[:, D_IN:, :]
    enc_b1r = enc_b1.reshape(T, 1, H)
    enc_b2r = enc_b2.reshape(T, 1, H)
    eu_wa = [eu_W1[l, :H] for l in range(L)]
    eu_wb = [eu_W1[l, H:2 * H] for l in range(L)]
    eu_wc = [eu_W1[l, 2 * H:] for l in range(L)]
    eu_b1r = [eu_b1[l].reshape(1, H) for l in range(L)]
    eu_b2r = [eu_b2[l].reshape(1, H) for l in range(L)]
    nu_w1h = [nu_W1[l, :H] for l in range(L)]
    nu_w1m = [nu_W1[l, H:] for l in range(L)]

    h, A, B = _enc_call(x, pe, enc_w1x, enc_w1p, enc_b1r, enc_W2, enc_b2r,
                        eu_wa[0], eu_wb[0])
    he, C = _eenc_call(edge_attr, ee_W1, ee_b1.reshape(1, H), ee_W2,
                       ee_b2.reshape(1, H), eu_wc[0], eu_b1r[0])

    for l in range(L):
        G = _sc_gather_add(A, B, src, dst)
        if l + 1 < L:
            he, C = _eupd_call(G, C, he, eu_W2[l], eu_b2r[l],
                               eu_wc[l + 1], eu_b1r[l + 1])
        else:
            he = _eupd_call(G, C, he, eu_W2[l], eu_b2r[l])
        m2 = _sc_scatter_add(he, dst)
        args = (h, m2[0], m2[1], nu_w1h[l], nu_w1m[l],
                nu_b1[l].reshape(1, H), nu_W2[l], nu_b2[l].reshape(1, H),
                fu_W1[l], fu_b1[l].reshape(1, H), fu_W2[l],
                fu_b2[l].reshape(1, H))
        if l + 1 < L:
            h, A, B = _nupd_call(*args, eu_wa[l + 1], eu_wb[l + 1])
        else:
            h = _nupd_call(*args)

    return _dec_call(x, h, dec_W1, dec_b1.reshape(T, 1, H), dec_W2,
                     dec_b2.reshape(T, 1, OUT))


# speedup vs baseline: 2.7926x; 1.5155x over previous
import functools

import jax
import jax.numpy as jnp
from jax import lax
from jax.experimental import pallas as pl
from jax.experimental.pallas import tpu as pltpu
from jax.experimental.pallas import tpu_sc as plsc

N = 10000
E = 320000
D_IN = 128
PE = 16
H = 64
OUT = 128
T = 3
L = 3

RN = 2000
RE = 4000

_NC = 2
_NS = 16
_NW = _NC * _NS
_EPW = E // _NW
_SUB = 125
_SPW = _EPW // _SUB
_GM = 2
_GNM = _SPW // _GM
_SM = 5
_SNM = _SPW // _SM
_NPS = N // _NS

@functools.cache
def _sc_mesh():
    return plsc.VectorSubcoreMesh(core_axis_name="c", subcore_axis_name="s",
                                  num_cores=_NC, num_subcores=_NS)


def _relu(v):
    return jnp.maximum(v, 0.0)


def _dot(a, b):
    return jnp.dot(a, b, preferred_element_type=jnp.float32)


def _full(a):
    return pl.BlockSpec(a.shape, lambda i: (0,) * a.ndim)


def _rows(shape):
    nd = len(shape)
    return pl.BlockSpec(shape, lambda i: (i,) + (0,) * (nd - 1))


def _enc_body(x_ref, pe_ref, w1x, w1p, b1, w2, b2, wa, wb,
              h_ref, a_ref, b_ref):
    xb = x_ref[...]
    peb = pe_ref[...]
    nt = jnp.clip(jnp.abs(xb[:, 2:3]).astype(jnp.int32), 0, T - 1)
    h = jnp.zeros((RN, H), jnp.float32)
    for t in range(T):
        h1 = _relu(_dot(xb, w1x[t]) + _dot(peb, w1p[t]) + b1[t])
        ht = _dot(h1, w2[t]) + b2[t]
        h = jnp.where(nt == t, ht, h)
    h_ref[...] = h
    a_ref[...] = _dot(h, wa[...])
    b_ref[...] = _dot(h, wb[...])


def _enc_call(x, pe, w1x, w1p, b1, w2, b2, wa, wb):
    return pl.pallas_call(
        _enc_body,
        grid=(N // RN,),
        in_specs=[_rows((RN, D_IN)), _rows((RN, PE))]
        + [_full(w) for w in (w1x, w1p, b1, w2, b2, wa, wb)],
        out_specs=[_rows((RN, H))] * 3,
        out_shape=[jax.ShapeDtypeStruct((N, H), jnp.float32)] * 3,
    )(x, pe, w1x, w1p, b1, w2, b2, wa, wb)


def _eenc_body(ea_ref, w1, b1, w2, b2, wc, bc, he_ref, c_ref):
    h1 = _relu(_dot(ea_ref[...], w1[...]) + b1[...])
    he = _dot(h1, w2[...]) + b2[...]
    he_ref[...] = he
    c_ref[...] = _dot(he, wc[...]) + bc[...]


def _eenc_call(ea, w1, b1, w2, b2, wc, bc):
    return pl.pallas_call(
        _eenc_body,
        grid=(E // RE,),
        in_specs=[_rows((RE, PE))] + [_full(w) for w in (w1, b1, w2, b2, wc, bc)],
        out_specs=[_rows((RE, H))] * 2,
        out_shape=[jax.ShapeDtypeStruct((E, H), jnp.float32)] * 2,
    )(ea, w1, b1, w2, b2, wc, bc)


def _eupd_body(g_ref, c_ref, he_ref, w2, b2, wcn, bcn, out_he, out_c):
    z = _relu(g_ref[...] + c_ref[...])
    hen = he_ref[...] + _dot(z, w2[...]) + b2[...]
    out_he[...] = hen
    out_c[...] = _dot(hen, wcn[...]) + bcn[...]


def _eupd_last_body(g_ref, c_ref, he_ref, w2, b2, out_he):
    z = _relu(g_ref[...] + c_ref[...])
    out_he[...] = he_ref[...] + _dot(z, w2[...]) + b2[...]


def _eupd_call(g, c, he, w2, b2, wcn=None, bcn=None):
    if wcn is None:
        return pl.pallas_call(
            _eupd_last_body,
            grid=(E // RE,),
            in_specs=[_rows((RE, H))] * 3 + [_full(w) for w in (w2, b2)],
            out_specs=[_rows((RE, H))],
            out_shape=[jax.ShapeDtypeStruct((E, H), jnp.float32)],
        )(g, c, he, w2, b2)[0]
    return pl.pallas_call(
        _eupd_body,
        grid=(E // RE,),
        in_specs=[_rows((RE, H))] * 3 + [_full(w) for w in (w2, b2, wcn, bcn)],
        out_specs=[_rows((RE, H))] * 2,
        out_shape=[jax.ShapeDtypeStruct((E, H), jnp.float32)] * 2,
    )(g, c, he, w2, b2, wcn, bcn)


def _nupd_body(h_ref, m0_ref, m1_ref, w1h, w1m, b1, w2, b2,
               f1, fb1, f2, fb2, wa, wb, h_out, a_out, b_out):
    h = h_ref[...]
    m = m0_ref[...] + m1_ref[...]
    u = _dot(_relu(_dot(h, w1h[...]) + _dot(m, w1m[...]) + b1[...]), w2[...]) + b2[...]
    local = h + u
    hn = _dot(_relu(_dot(local, f1[...]) + fb1[...]), f2[...]) + fb2[...]
    h_out[...] = hn
    a_out[...] = _dot(hn, wa[...])
    b_out[...] = _dot(hn, wb[...])


def _nupd_last_body(h_ref, m0_ref, m1_ref, w1h, w1m, b1, w2, b2,
                    f1, fb1, f2, fb2, h_out):
    h = h_ref[...]
    m = m0_ref[...] + m1_ref[...]
    u = _dot(_relu(_dot(h, w1h[...]) + _dot(m, w1m[...]) + b1[...]), w2[...]) + b2[...]
    local = h + u
    h_out[...] = _dot(_relu(_dot(local, f1[...]) + fb1[...]), f2[...]) + fb2[...]


def _nupd_call(h, m0, m1, w1h, w1m, b1, w2, b2, f1, fb1, f2, fb2,
               wa=None, wb=None):
    if wa is None:
        return pl.pallas_call(
            _nupd_last_body,
            grid=(N // RN,),
            in_specs=[_rows((RN, H))] * 3
            + [_full(w) for w in (w1h, w1m, b1, w2, b2, f1, fb1, f2, fb2)],
            out_specs=[_rows((RN, H))],
            out_shape=[jax.ShapeDtypeStruct((N, H), jnp.float32)],
        )(h, m0, m1, w1h, w1m, b1, w2, b2, f1, fb1, f2, fb2)[0]
    return pl.pallas_call(
        _nupd_body,
        grid=(N // RN,),
        in_specs=[_rows((RN, H))] * 3
        + [_full(w) for w in (w1h, w1m, b1, w2, b2, f1, fb1, f2, fb2, wa, wb)],
        out_specs=[_rows((RN, H))] * 3,
        out_shape=[jax.ShapeDtypeStruct((N, H), jnp.float32)] * 3,
    )(h, m0, m1, w1h, w1m, b1, w2, b2, f1, fb1, f2, fb2, wa, wb)


def _dec_body(x_ref, h_ref, w1, b1, w2, b2, out_ref):
    nt = jnp.clip(jnp.abs(x_ref[:, 2:3]).astype(jnp.int32), 0, T - 1)
    h = h_ref[...]
    out = jnp.zeros((RN, OUT), jnp.float32)
    for t in range(T):
        ot = _dot(_relu(_dot(h, w1[t]) + b1[t]), w2[t]) + b2[t]
        out = jnp.where(nt == t, ot, out)
    out_ref[...] = out


def _dec_call(x, h, w1, b1, w2, b2):
    return pl.pallas_call(
        _dec_body,
        grid=(N // RN,),
        in_specs=[_rows((RN, D_IN)), _rows((RN, H))]
        + [_full(w) for w in (w1, b1, w2, b2)],
        out_specs=_rows((RN, OUT)),
        out_shape=jax.ShapeDtypeStruct((N, OUT), jnp.float32),
    )(x, h, w1, b1, w2, b2)


@functools.cache
def _sc_gather_add_kernel():
    return pl.kernel(
        _sc_gather_add_body,
        out_type=jax.ShapeDtypeStruct((E, H), jnp.float32),
        mesh=_sc_mesh(),
        scratch_types=[
            pltpu.VMEM((_SPW, _SUB), jnp.int32),
            pltpu.VMEM((_SPW, _SUB), jnp.int32),
            [pltpu.VMEM((_SUB, H), jnp.float32)] * 4,
            [pltpu.VMEM((_SUB, H), jnp.float32)] * 4,
            [pltpu.VMEM((_GM * _SUB, H), jnp.float32)] * 2,
            [pltpu.SemaphoreType.DMA] * 4,
            [pltpu.SemaphoreType.DMA] * 2,
        ],
        compiler_params=pltpu.CompilerParams(use_tc_tiling_on_sc=False),
    )


def _sc_gather_add(a, b, src2, dst2):
    return _sc_gather_add_kernel()(a, b, src2, dst2)


def _sc_gather_add_body(a_hbm, b_hbm, src_hbm, dst_hbm, out_hbm,
                        sidx, didx, ras, rbs, obs, sgs, sws):
    wid = lax.axis_index("s") * _NC + lax.axis_index("c")
    pltpu.sync_copy(src_hbm.at[pl.ds(wid * _SPW, _SPW)], sidx)
    pltpu.sync_copy(dst_hbm.at[pl.ds(wid * _SPW, _SPW)], didx)

    def issue(t, s):
        pltpu.async_copy(a_hbm.at[sidx.at[t]], ras[s], sgs[s])
        pltpu.async_copy(b_hbm.at[didx.at[t]], rbs[s], sgs[s])

    def wait_gather(s):
        pltpu.make_async_copy(a_hbm.at[sidx.at[0]], ras[s], sgs[s]).wait()
        pltpu.make_async_copy(b_hbm.at[didx.at[0]], rbs[s], sgs[s]).wait()

    for s in range(4):
        issue(s, s)

    def two_megas(ii, carry):
        for mm in range(2):
            m = 2 * ii + mm
            ob = obs[mm]
            @pl.when(m >= 2)
            def _():
                pltpu.make_async_copy(
                    ob, out_hbm.at[pl.ds(0, _GM * _SUB)], sws[mm]).wait()

            for k in range(_GM):
                s = 2 * mm + k
                t = _GM * m + k
                wait_gather(s)
                ra, rb = ras[s], rbs[s]

                def row(r, c2, ra=ra, rb=rb, ob=ob, k=k):
                    for cc in range(H // 16):
                        sl = pl.ds(cc * 16, 16)
                        ob[k * _SUB + r, sl] = ra[r, sl] + rb[r, sl]
                    return c2

                lax.fori_loop(0, _SUB, row, 0)
                nxt = t + 4

                @pl.when(nxt < _SPW)
                def _():
                    issue(nxt, s)

            pltpu.async_copy(
                ob,
                out_hbm.at[pl.ds(wid * _EPW + m * _GM * _SUB, _GM * _SUB)],
                sws[mm])
        return carry

    lax.fori_loop(0, _GNM // 2, two_megas, 0)
    for mm in range(2):
        pltpu.make_async_copy(
            obs[mm], out_hbm.at[pl.ds(0, _GM * _SUB)], sws[mm]).wait()


@functools.cache
def _sc_scatter_add_kernel():
    return pl.kernel(
        _sc_scatter_add_body,
        out_type=jax.ShapeDtypeStruct((_NC, N, H), jnp.float32),
        mesh=_sc_mesh(),
        scratch_types=[
            pltpu.VMEM((_SPW, _SUB), jnp.int32),
            [pltpu.VMEM((_SM * _SUB, H), jnp.float32)] * 2,
            pltpu.VMEM_SHARED((N, H), jnp.float32),
            [pltpu.SemaphoreType.DMA] * 2,
            [pltpu.SemaphoreType.DMA] * 2,
        ],
        compiler_params=pltpu.CompilerParams(use_tc_tiling_on_sc=False),
    )


def _sc_scatter_add(he, dst2):
    return _sc_scatter_add_kernel()(he, dst2)


def _sc_scatter_add_body(he_hbm, dst_hbm, out_hbm, didx, rs, shared, lsems, ssems):
    cid = lax.axis_index("c")
    sid = lax.axis_index("s")
    wid = sid * _NC + cid
    base = wid * _EPW
    mrows = _SM * _SUB

    def zrow(r, c2):
        for cc in range(H // 16):
            rs[0][r, pl.ds(cc * 16, 16)] = jnp.zeros((16,), jnp.float32)
        return c2

    lax.fori_loop(0, _NPS, zrow, 0)
    pltpu.sync_copy(rs[0], shared.at[pl.ds(sid * _NPS, _NPS)])
    pltpu.sync_copy(dst_hbm.at[pl.ds(wid * _SPW, _SPW)], didx)
    plsc.subcore_barrier()

    def load(m, s):
        pltpu.async_copy(he_hbm.at[pl.ds(base + m * mrows, mrows)],
                         rs[s], lsems[s])

    def wait_load(s):
        pltpu.make_async_copy(he_hbm.at[pl.ds(0, mrows)], rs[s],
                              lsems[s]).wait()

    def wait_scatter(s):
        pltpu.make_async_copy(rs[s], shared.at[pl.ds(0, mrows)],
                              ssems[s]).wait()

    load(0, 0)

    def two_megas(ii, carry):
        for mm in range(2):
            m = 2 * ii + mm
            wait_load(mm)

            @pl.when(m + 1 < _SNM)
            def _():
                @pl.when(m >= 1)
                def _():
                    wait_scatter(1 - mm)

                load(m + 1, 1 - mm)

            for b in range(_SM):
                t = _SM * m + b
                pltpu.async_copy(rs[mm].at[pl.ds(b * _SUB, _SUB)],
                                 shared.at[didx.at[t]], ssems[mm], add=True)
        return carry

    lax.fori_loop(0, _SNM // 2, two_megas, 0)
    wait_scatter(0)
    wait_scatter(1)
    plsc.subcore_barrier()
    pltpu.sync_copy(shared.at[pl.ds(sid * _NPS, _NPS)],
                    out_hbm.at[cid, pl.ds(sid * _NPS, _NPS)])


def kernel(x, edge_attr, pe, edge_index,
           enc_W1, enc_b1, enc_W2, enc_b2,
           ee_W1, ee_b1, ee_W2, ee_b2,
           eu_W1, eu_b1, eu_W2, eu_b2,
           nu_W1, nu_b1, nu_W2, nu_b2,
           fu_W1, fu_b1, fu_W2, fu_b2,
           dec_W1, dec_b1, dec_W2, dec_b2):
    src2 = edge_index[0].astype(jnp.int32).reshape(E // _SUB, _SUB)
    dst2 = edge_index[1].astype(jnp.int32).reshape(E // _SUB, _SUB)

    enc_w1x = enc_W1[:, :D_IN, :]
    enc_w1p = enc_W1[:, D_IN:, :]
    enc_b1r = enc_b1.reshape(T, 1, H)
    enc_b2r = enc_b2.reshape(T, 1, H)
    eu_wa = [eu_W1[l, :H] for l in range(L)]
    eu_wb = [eu_W1[l, H:2 * H] for l in range(L)]
    eu_wc = [eu_W1[l, 2 * H:] for l in range(L)]
    eu_b1r = [eu_b1[l].reshape(1, H) for l in range(L)]
    eu_b2r = [eu_b2[l].reshape(1, H) for l in range(L)]
    nu_w1h = [nu_W1[l, :H] for l in range(L)]
    nu_w1m = [nu_W1[l, H:] for l in range(L)]

    h, A, B = _enc_call(x, pe, enc_w1x, enc_w1p, enc_b1r, enc_W2, enc_b2r,
                        eu_wa[0], eu_wb[0])
    he, C = _eenc_call(edge_attr, ee_W1, ee_b1.reshape(1, H), ee_W2,
                       ee_b2.reshape(1, H), eu_wc[0], eu_b1r[0])

    for l in range(L):
        G = _sc_gather_add(A, B, src2, dst2)
        if l + 1 < L:
            he, C = _eupd_call(G, C, he, eu_W2[l], eu_b2r[l],
                               eu_wc[l + 1], eu_b1r[l + 1])
        else:
            he = _eupd_call(G, C, he, eu_W2[l], eu_b2r[l])
        m2 = _sc_scatter_add(he, dst2)
        args = (h, m2[0], m2[1], nu_w1h[l], nu_w1m[l],
                nu_b1[l].reshape(1, H), nu_W2[l], nu_b2[l].reshape(1, H),
                fu_W1[l], fu_b1[l].reshape(1, H), fu_W2[l],
                fu_b2[l].reshape(1, H))
        if l + 1 < L:
            h, A, B = _nupd_call(*args, eu_wa[l + 1], eu_wb[l + 1])
        else:
            h = _nupd_call(*args)

    return _dec_call(x, h, dec_W1, dec_b1.reshape(T, 1, H), dec_W2,
                     dec_b2.reshape(T, 1, OUT))


# speedup vs baseline: 3.1533x; 1.1291x over previous
import functools

import jax
import jax.numpy as jnp
from jax import lax
from jax.experimental import pallas as pl
from jax.experimental.pallas import tpu as pltpu
from jax.experimental.pallas import tpu_sc as plsc

N = 10000
E = 320000
D_IN = 128
PE = 16
H = 64
OUT = 128
T = 3
L = 3

RN = 2000
RE = 4000

_NC = 2
_NS = 16
_NW = _NC * _NS
_EPW = E // _NW
_SUB = 125
_SPW = _EPW // _SUB
_GM = 2
_GNM = _SPW // _GM
_SM = 5
_SNM = _SPW // _SM
_NPS = N // _NS

@functools.cache
def _sc_mesh():
    return plsc.VectorSubcoreMesh(core_axis_name="c", subcore_axis_name="s",
                                  num_cores=_NC, num_subcores=_NS)


def _relu(v):
    return jnp.maximum(v, 0.0)


def _dot(a, b):
    return jnp.dot(a, b, preferred_element_type=jnp.float32)


def _full(a):
    return pl.BlockSpec(a.shape, lambda i: (0,) * a.ndim)


def _rows(shape):
    nd = len(shape)
    return pl.BlockSpec(shape, lambda i: (i,) + (0,) * (nd - 1))


def _enc_body(x_ref, pe_ref, w1x, w1p, b1, w2, b2, wa, wb,
              h_ref, a_ref, b_ref):
    xb = x_ref[...]
    peb = pe_ref[...]
    nt = jnp.clip(jnp.abs(xb[:, 2:3]).astype(jnp.int32), 0, T - 1)
    h = jnp.zeros((RN, H), jnp.float32)
    for t in range(T):
        h1 = _relu(_dot(xb, w1x[t]) + _dot(peb, w1p[t]) + b1[t])
        ht = _dot(h1, w2[t]) + b2[t]
        h = jnp.where(nt == t, ht, h)
    h_ref[...] = h
    a_ref[...] = _dot(h, wa[...])
    b_ref[...] = _dot(h, wb[...])


def _enc_call(x, pe, w1x, w1p, b1, w2, b2, wa, wb):
    return pl.pallas_call(
        _enc_body,
        grid=(N // RN,),
        in_specs=[_rows((RN, D_IN)), _rows((RN, PE))]
        + [_full(w) for w in (w1x, w1p, b1, w2, b2, wa, wb)],
        out_specs=[_rows((RN, H))] * 3,
        out_shape=[jax.ShapeDtypeStruct((N, H), jnp.float32)] * 3,
    )(x, pe, w1x, w1p, b1, w2, b2, wa, wb)


def _eenc_body(ea_ref, w1, b1, w2, b2, he_ref):
    h1 = _relu(_dot(ea_ref[...], w1[...]) + b1[...])
    he_ref[...] = _dot(h1, w2[...]) + b2[...]


def _eenc_call(ea, w1, b1, w2, b2):
    return pl.pallas_call(
        _eenc_body,
        grid=(E // RE,),
        in_specs=[_rows((RE, PE))] + [_full(w) for w in (w1, b1, w2, b2)],
        out_specs=_rows((RE, H)),
        out_shape=jax.ShapeDtypeStruct((E, H), jnp.float32),
    )(ea, w1, b1, w2, b2)


def _eupd_body(g_ref, he_ref, wc, bc, w2, b2, out_he):
    he = he_ref[...]
    z = _relu(g_ref[...] + _dot(he, wc[...]) + bc[...])
    out_he[...] = he + _dot(z, w2[...]) + b2[...]


def _eupd_call(g, he, wc, bc, w2, b2):
    return pl.pallas_call(
        _eupd_body,
        grid=(E // RE,),
        in_specs=[_rows((RE, H))] * 2 + [_full(w) for w in (wc, bc, w2, b2)],
        out_specs=_rows((RE, H)),
        out_shape=jax.ShapeDtypeStruct((E, H), jnp.float32),
    )(g, he, wc, bc, w2, b2)


def _nupd_body(h_ref, m0_ref, m1_ref, w1h, w1m, b1, w2, b2,
               f1, fb1, f2, fb2, wa, wb, h_out, a_out, b_out):
    h = h_ref[...]
    m = m0_ref[...] + m1_ref[...]
    u = _dot(_relu(_dot(h, w1h[...]) + _dot(m, w1m[...]) + b1[...]), w2[...]) + b2[...]
    local = h + u
    hn = _dot(_relu(_dot(local, f1[...]) + fb1[...]), f2[...]) + fb2[...]
    h_out[...] = hn
    a_out[...] = _dot(hn, wa[...])
    b_out[...] = _dot(hn, wb[...])


def _nupd_last_body(h_ref, m0_ref, m1_ref, w1h, w1m, b1, w2, b2,
                    f1, fb1, f2, fb2, h_out):
    h = h_ref[...]
    m = m0_ref[...] + m1_ref[...]
    u = _dot(_relu(_dot(h, w1h[...]) + _dot(m, w1m[...]) + b1[...]), w2[...]) + b2[...]
    local = h + u
    h_out[...] = _dot(_relu(_dot(local, f1[...]) + fb1[...]), f2[...]) + fb2[...]


def _nupd_call(h, m0, m1, w1h, w1m, b1, w2, b2, f1, fb1, f2, fb2,
               wa=None, wb=None):
    if wa is None:
        return pl.pallas_call(
            _nupd_last_body,
            grid=(N // RN,),
            in_specs=[_rows((RN, H))] * 3
            + [_full(w) for w in (w1h, w1m, b1, w2, b2, f1, fb1, f2, fb2)],
            out_specs=[_rows((RN, H))],
            out_shape=[jax.ShapeDtypeStruct((N, H), jnp.float32)],
        )(h, m0, m1, w1h, w1m, b1, w2, b2, f1, fb1, f2, fb2)[0]
    return pl.pallas_call(
        _nupd_body,
        grid=(N // RN,),
        in_specs=[_rows((RN, H))] * 3
        + [_full(w) for w in (w1h, w1m, b1, w2, b2, f1, fb1, f2, fb2, wa, wb)],
        out_specs=[_rows((RN, H))] * 3,
        out_shape=[jax.ShapeDtypeStruct((N, H), jnp.float32)] * 3,
    )(h, m0, m1, w1h, w1m, b1, w2, b2, f1, fb1, f2, fb2, wa, wb)


def _dec_body(x_ref, h_ref, w1, b1, w2, b2, out_ref):
    nt = jnp.clip(jnp.abs(x_ref[:, 2:3]).astype(jnp.int32), 0, T - 1)
    h = h_ref[...]
    out = jnp.zeros((RN, OUT), jnp.float32)
    for t in range(T):
        ot = _dot(_relu(_dot(h, w1[t]) + b1[t]), w2[t]) + b2[t]
        out = jnp.where(nt == t, ot, out)
    out_ref[...] = out


def _dec_call(x, h, w1, b1, w2, b2):
    return pl.pallas_call(
        _dec_body,
        grid=(N // RN,),
        in_specs=[_rows((RN, D_IN)), _rows((RN, H))]
        + [_full(w) for w in (w1, b1, w2, b2)],
        out_specs=_rows((RN, OUT)),
        out_shape=jax.ShapeDtypeStruct((N, OUT), jnp.float32),
    )(x, h, w1, b1, w2, b2)


@functools.cache
def _sc_gather_add_kernel():
    return pl.kernel(
        _sc_gather_add_body,
        out_type=jax.ShapeDtypeStruct((E, H), jnp.float32),
        mesh=_sc_mesh(),
        scratch_types=[
            pltpu.VMEM((_SPW, _SUB), jnp.int32),
            pltpu.VMEM((_SPW, _SUB), jnp.int32),
            [pltpu.VMEM((_SUB, H), jnp.float32)] * 4,
            [pltpu.VMEM((_SUB, H), jnp.float32)] * 4,
            [pltpu.VMEM((_GM * _SUB, H), jnp.float32)] * 2,
            [pltpu.SemaphoreType.DMA] * 4,
            [pltpu.SemaphoreType.DMA] * 2,
        ],
        compiler_params=pltpu.CompilerParams(use_tc_tiling_on_sc=False),
    )


def _sc_gather_add(a, b, src2, dst2):
    return _sc_gather_add_kernel()(a, b, src2, dst2)


def _sc_gather_add_body(a_hbm, b_hbm, src_hbm, dst_hbm, out_hbm,
                        sidx, didx, ras, rbs, obs, sgs, sws):
    wid = lax.axis_index("s") * _NC + lax.axis_index("c")
    pltpu.sync_copy(src_hbm.at[pl.ds(wid * _SPW, _SPW)], sidx)
    pltpu.sync_copy(dst_hbm.at[pl.ds(wid * _SPW, _SPW)], didx)

    def issue(t, s):
        pltpu.async_copy(a_hbm.at[sidx.at[t]], ras[s], sgs[s])
        pltpu.async_copy(b_hbm.at[didx.at[t]], rbs[s], sgs[s])

    def wait_gather(s):
        pltpu.make_async_copy(a_hbm.at[sidx.at[0]], ras[s], sgs[s]).wait()
        pltpu.make_async_copy(b_hbm.at[didx.at[0]], rbs[s], sgs[s]).wait()

    for s in range(4):
        issue(s, s)

    def two_megas(ii, carry):
        for mm in range(2):
            m = 2 * ii + mm
            ob = obs[mm]
            @pl.when(m >= 2)
            def _():
                pltpu.make_async_copy(
                    ob, out_hbm.at[pl.ds(0, _GM * _SUB)], sws[mm]).wait()

            for k in range(_GM):
                s = 2 * mm + k
                t = _GM * m + k
                wait_gather(s)
                ra, rb = ras[s], rbs[s]

                def row(r, c2, ra=ra, rb=rb, ob=ob, k=k):
                    for cc in range(H // 16):
                        sl = pl.ds(cc * 16, 16)
                        ob[k * _SUB + r, sl] = ra[r, sl] + rb[r, sl]
                    return c2

                lax.fori_loop(0, _SUB, row, 0)
                nxt = t + 4

                @pl.when(nxt < _SPW)
                def _():
                    issue(nxt, s)

            pltpu.async_copy(
                ob,
                out_hbm.at[pl.ds(wid * _EPW + m * _GM * _SUB, _GM * _SUB)],
                sws[mm])
        return carry

    lax.fori_loop(0, _GNM // 2, two_megas, 0)
    for mm in range(2):
        pltpu.make_async_copy(
            obs[mm], out_hbm.at[pl.ds(0, _GM * _SUB)], sws[mm]).wait()


@functools.cache
def _sc_scatter_add_kernel():
    return pl.kernel(
        _sc_scatter_add_body,
        out_type=jax.ShapeDtypeStruct((_NC, N, H), jnp.float32),
        mesh=_sc_mesh(),
        scratch_types=[
            pltpu.VMEM((_SPW, _SUB), jnp.int32),
            [pltpu.VMEM((_SM * _SUB, H), jnp.float32)] * 2,
            pltpu.VMEM_SHARED((N, H), jnp.float32),
            [pltpu.SemaphoreType.DMA] * 2,
            [pltpu.SemaphoreType.DMA] * 2,
        ],
        compiler_params=pltpu.CompilerParams(use_tc_tiling_on_sc=False),
    )


def _sc_scatter_add(he, dst2):
    return _sc_scatter_add_kernel()(he, dst2)


def _sc_scatter_add_body(he_hbm, dst_hbm, out_hbm, didx, rs, shared, lsems, ssems):
    cid = lax.axis_index("c")
    sid = lax.axis_index("s")
    wid = sid * _NC + cid
    base = wid * _EPW
    mrows = _SM * _SUB

    def zrow(r, c2):
        for cc in range(H // 16):
            rs[0][r, pl.ds(cc * 16, 16)] = jnp.zeros((16,), jnp.float32)
        return c2

    lax.fori_loop(0, _NPS, zrow, 0)
    pltpu.sync_copy(rs[0], shared.at[pl.ds(sid * _NPS, _NPS)])
    pltpu.sync_copy(dst_hbm.at[pl.ds(wid * _SPW, _SPW)], didx)
    plsc.subcore_barrier()

    def load(m, s):
        pltpu.async_copy(he_hbm.at[pl.ds(base + m * mrows, mrows)],
                         rs[s], lsems[s])

    def wait_load(s):
        pltpu.make_async_copy(he_hbm.at[pl.ds(0, mrows)], rs[s],
                              lsems[s]).wait()

    def wait_scatter(s):
        pltpu.make_async_copy(rs[s], shared.at[pl.ds(0, mrows)],
                              ssems[s]).wait()

    load(0, 0)

    def two_megas(ii, carry):
        for mm in range(2):
            m = 2 * ii + mm
            wait_load(mm)

            @pl.when(m + 1 < _SNM)
            def _():
                @pl.when(m >= 1)
                def _():
                    wait_scatter(1 - mm)

                load(m + 1, 1 - mm)

            for b in range(_SM):
                t = _SM * m + b
                pltpu.async_copy(rs[mm].at[pl.ds(b * _SUB, _SUB)],
                                 shared.at[didx.at[t]], ssems[mm], add=True)
        return carry

    lax.fori_loop(0, _SNM // 2, two_megas, 0)
    wait_scatter(0)
    wait_scatter(1)
    plsc.subcore_barrier()
    pltpu.sync_copy(shared.at[pl.ds(sid * _NPS, _NPS)],
                    out_hbm.at[cid, pl.ds(sid * _NPS, _NPS)])


def kernel(x, edge_attr, pe, edge_index,
           enc_W1, enc_b1, enc_W2, enc_b2,
           ee_W1, ee_b1, ee_W2, ee_b2,
           eu_W1, eu_b1, eu_W2, eu_b2,
           nu_W1, nu_b1, nu_W2, nu_b2,
           fu_W1, fu_b1, fu_W2, fu_b2,
           dec_W1, dec_b1, dec_W2, dec_b2):
    src2 = edge_index[0].astype(jnp.int32).reshape(E // _SUB, _SUB)
    dst2 = edge_index[1].astype(jnp.int32).reshape(E // _SUB, _SUB)

    enc_w1x = enc_W1[:, :D_IN, :]
    enc_w1p = enc_W1[:, D_IN:, :]
    enc_b1r = enc_b1.reshape(T, 1, H)
    enc_b2r = enc_b2.reshape(T, 1, H)
    eu_wa = [eu_W1[l, :H] for l in range(L)]
    eu_wb = [eu_W1[l, H:2 * H] for l in range(L)]
    eu_wc = [eu_W1[l, 2 * H:] for l in range(L)]
    eu_b1r = [eu_b1[l].reshape(1, H) for l in range(L)]
    eu_b2r = [eu_b2[l].reshape(1, H) for l in range(L)]
    nu_w1h = [nu_W1[l, :H] for l in range(L)]
    nu_w1m = [nu_W1[l, H:] for l in range(L)]

    h, A, B = _enc_call(x, pe, enc_w1x, enc_w1p, enc_b1r, enc_W2, enc_b2r,
                        eu_wa[0], eu_wb[0])
    he = _eenc_call(edge_attr, ee_W1, ee_b1.reshape(1, H), ee_W2,
                    ee_b2.reshape(1, H))

    for l in range(L):
        G = _sc_gather_add(A, B, src2, dst2)
        he = _eupd_call(G, he, eu_wc[l], eu_b1r[l], eu_W2[l], eu_b2r[l])
        m2 = _sc_scatter_add(he, dst2)
        args = (h, m2[0], m2[1], nu_w1h[l], nu_w1m[l],
                nu_b1[l].reshape(1, H), nu_W2[l], nu_b2[l].reshape(1, H),
                fu_W1[l], fu_b1[l].reshape(1, H), fu_W2[l],
                fu_b2[l].reshape(1, H))
        if l + 1 < L:
            h, A, B = _nupd_call(*args, eu_wa[l + 1], eu_wb[l + 1])
        else:
            h = _nupd_call(*args)

    return _dec_call(x, h, dec_W1, dec_b1.reshape(T, 1, H), dec_W2,
                     dec_b2.reshape(T, 1, OUT))


# speedup vs baseline: 3.2515x; 1.0311x over previous
import functools

import jax
import jax.numpy as jnp
from jax import lax
from jax.experimental import pallas as pl
from jax.experimental.pallas import tpu as pltpu
from jax.experimental.pallas import tpu_sc as plsc

N = 10000
E = 320000
D_IN = 128
PE = 16
H = 64
OUT = 128
T = 3
L = 3

RN = 2000
RE = 4000

_NC = 2
_NS = 16
_NW = _NC * _NS
_SUB = 125
_GM = 2
_SM = 5
_NPS = N // _NS
EH = E // 2

@functools.cache
def _sc_mesh():
    return plsc.VectorSubcoreMesh(core_axis_name="c", subcore_axis_name="s",
                                  num_cores=_NC, num_subcores=_NS)


def _relu(v):
    return jnp.maximum(v, 0.0)


def _dot(a, b):
    return jnp.dot(a, b, preferred_element_type=jnp.float32)


def _full(a):
    return pl.BlockSpec(a.shape, lambda i: (0,) * a.ndim)


def _rows(shape):
    nd = len(shape)
    return pl.BlockSpec(shape, lambda i: (i,) + (0,) * (nd - 1))


def _enc_body(x_ref, pe_ref, w1x, w1p, b1, w2, b2, wa, wb,
              h_ref, a_ref, b_ref):
    xb = x_ref[...]
    peb = pe_ref[...]
    nt = jnp.clip(jnp.abs(xb[:, 2:3]).astype(jnp.int32), 0, T - 1)
    h = jnp.zeros((RN, H), jnp.float32)
    for t in range(T):
        h1 = _relu(_dot(xb, w1x[t]) + _dot(peb, w1p[t]) + b1[t])
        ht = _dot(h1, w2[t]) + b2[t]
        h = jnp.where(nt == t, ht, h)
    h_ref[...] = h
    a_ref[...] = _dot(h, wa[...])
    b_ref[...] = _dot(h, wb[...])


def _enc_call(x, pe, w1x, w1p, b1, w2, b2, wa, wb):
    return pl.pallas_call(
        _enc_body,
        grid=(N // RN,),
        in_specs=[_rows((RN, D_IN)), _rows((RN, PE))]
        + [_full(w) for w in (w1x, w1p, b1, w2, b2, wa, wb)],
        out_specs=[_rows((RN, H))] * 3,
        out_shape=[jax.ShapeDtypeStruct((N, H), jnp.float32)] * 3,
    )(x, pe, w1x, w1p, b1, w2, b2, wa, wb)


def _eenc_body(ea_ref, w1, b1, w2, b2, he_ref):
    h1 = _relu(_dot(ea_ref[...], w1[...]) + b1[...])
    he_ref[...] = _dot(h1, w2[...]) + b2[...]


def _eenc_call(ea, off, w1, b1, w2, b2):
    return pl.pallas_call(
        _eenc_body,
        grid=(EH // RE,),
        in_specs=[pl.BlockSpec((RE, PE), lambda i: (i + off, 0))]
        + [_full(w) for w in (w1, b1, w2, b2)],
        out_specs=_rows((RE, H)),
        out_shape=jax.ShapeDtypeStruct((EH, H), jnp.float32),
    )(ea, w1, b1, w2, b2)


def _eupd_body(g_ref, he_ref, wc, bc, w2, b2, out_he):
    he = he_ref[...]
    z = _relu(g_ref[...] + _dot(he, wc[...]) + bc[...])
    out_he[...] = he + _dot(z, w2[...]) + b2[...]


def _eupd_call(g, he, wc, bc, w2, b2):
    ne = g.shape[0]
    return pl.pallas_call(
        _eupd_body,
        grid=(ne // RE,),
        in_specs=[_rows((RE, H))] * 2 + [_full(w) for w in (wc, bc, w2, b2)],
        out_specs=_rows((RE, H)),
        out_shape=jax.ShapeDtypeStruct((ne, H), jnp.float32),
    )(g, he, wc, bc, w2, b2)


def _nupd_body(h_ref, ma_ref, mb_ref, w1h, w1m, b1, w2, b2,
               f1, fb1, f2, fb2, wa, wb, h_out, a_out, b_out):
    h = h_ref[...]
    m = ma_ref[0] + ma_ref[1] + mb_ref[0] + mb_ref[1]
    u = _dot(_relu(_dot(h, w1h[...]) + _dot(m, w1m[...]) + b1[...]), w2[...]) + b2[...]
    local = h + u
    hn = _dot(_relu(_dot(local, f1[...]) + fb1[...]), f2[...]) + fb2[...]
    h_out[...] = hn
    a_out[...] = _dot(hn, wa[...])
    b_out[...] = _dot(hn, wb[...])


def _nupd_last_body(h_ref, ma_ref, mb_ref, w1h, w1m, b1, w2, b2,
                    f1, fb1, f2, fb2, h_out):
    h = h_ref[...]
    m = ma_ref[0] + ma_ref[1] + mb_ref[0] + mb_ref[1]
    u = _dot(_relu(_dot(h, w1h[...]) + _dot(m, w1m[...]) + b1[...]), w2[...]) + b2[...]
    local = h + u
    h_out[...] = _dot(_relu(_dot(local, f1[...]) + fb1[...]), f2[...]) + fb2[...]


_mspec = pl.BlockSpec((_NC, RN, H), lambda i: (0, i, 0))


def _nupd_call(h, ma, mb, w1h, w1m, b1, w2, b2, f1, fb1, f2, fb2,
               wa=None, wb=None):
    if wa is None:
        return pl.pallas_call(
            _nupd_last_body,
            grid=(N // RN,),
            in_specs=[_rows((RN, H)), _mspec, _mspec]
            + [_full(w) for w in (w1h, w1m, b1, w2, b2, f1, fb1, f2, fb2)],
            out_specs=[_rows((RN, H))],
            out_shape=[jax.ShapeDtypeStruct((N, H), jnp.float32)],
        )(h, ma, mb, w1h, w1m, b1, w2, b2, f1, fb1, f2, fb2)[0]
    return pl.pallas_call(
        _nupd_body,
        grid=(N // RN,),
        in_specs=[_rows((RN, H)), _mspec, _mspec]
        + [_full(w) for w in (w1h, w1m, b1, w2, b2, f1, fb1, f2, fb2, wa, wb)],
        out_specs=[_rows((RN, H))] * 3,
        out_shape=[jax.ShapeDtypeStruct((N, H), jnp.float32)] * 3,
    )(h, ma, mb, w1h, w1m, b1, w2, b2, f1, fb1, f2, fb2, wa, wb)


def _dec_body(x_ref, h_ref, w1, b1, w2, b2, out_ref):
    nt = jnp.clip(jnp.abs(x_ref[:, 2:3]).astype(jnp.int32), 0, T - 1)
    h = h_ref[...]
    out = jnp.zeros((RN, OUT), jnp.float32)
    for t in range(T):
        ot = _dot(_relu(_dot(h, w1[t]) + b1[t]), w2[t]) + b2[t]
        out = jnp.where(nt == t, ot, out)
    out_ref[...] = out


def _dec_call(x, h, w1, b1, w2, b2):
    return pl.pallas_call(
        _dec_body,
        grid=(N // RN,),
        in_specs=[_rows((RN, D_IN)), _rows((RN, H))]
        + [_full(w) for w in (w1, b1, w2, b2)],
        out_specs=_rows((RN, OUT)),
        out_shape=jax.ShapeDtypeStruct((N, OUT), jnp.float32),
    )(x, h, w1, b1, w2, b2)


@functools.cache
def _sc_gather_add_kernel(ne):
    spw = ne // _NW // _SUB
    return pl.kernel(
        functools.partial(_sc_gather_add_body, ne),
        out_type=jax.ShapeDtypeStruct((ne, H), jnp.float32),
        mesh=_sc_mesh(),
        scratch_types=[
            pltpu.VMEM((spw, _SUB), jnp.int32),
            pltpu.VMEM((spw, _SUB), jnp.int32),
            [pltpu.VMEM((_SUB, H), jnp.float32)] * 4,
            [pltpu.VMEM((_SUB, H), jnp.float32)] * 4,
            [pltpu.VMEM((_GM * _SUB, H), jnp.float32)] * 2,
            [pltpu.SemaphoreType.DMA] * 4,
            [pltpu.SemaphoreType.DMA] * 2,
        ],
        compiler_params=pltpu.CompilerParams(use_tc_tiling_on_sc=False),
    )


def _sc_gather_add(a, b, src2, dst2):
    ne = src2.shape[0] * _SUB
    return _sc_gather_add_kernel(ne)(a, b, src2, dst2)


def _sc_gather_add_body(ne, a_hbm, b_hbm, src_hbm, dst_hbm, out_hbm,
                        sidx, didx, ras, rbs, obs, sgs, sws):
    epw = ne // _NW
    spw = epw // _SUB
    gnm = spw // _GM
    wid = lax.axis_index("s") * _NC + lax.axis_index("c")
    pltpu.sync_copy(src_hbm.at[pl.ds(wid * spw, spw)], sidx)
    pltpu.sync_copy(dst_hbm.at[pl.ds(wid * spw, spw)], didx)

    def issue(t, s):
        pltpu.async_copy(a_hbm.at[sidx.at[t]], ras[s], sgs[s])
        pltpu.async_copy(b_hbm.at[didx.at[t]], rbs[s], sgs[s])

    def wait_gather(s):
        pltpu.make_async_copy(a_hbm.at[sidx.at[0]], ras[s], sgs[s]).wait()
        pltpu.make_async_copy(b_hbm.at[didx.at[0]], rbs[s], sgs[s]).wait()

    for s in range(4):
        issue(s, s)

    def two_megas(ii, carry):
        for mm in range(2):
            m = 2 * ii + mm
            ob = obs[mm]
            @pl.when(m >= 2)
            def _():
                pltpu.make_async_copy(
                    ob, out_hbm.at[pl.ds(0, _GM * _SUB)], sws[mm]).wait()

            for k in range(_GM):
                s = 2 * mm + k
                t = _GM * m + k
                wait_gather(s)
                ra, rb = ras[s], rbs[s]

                def row(r, c2, ra=ra, rb=rb, ob=ob, k=k):
                    for cc in range(H // 16):
                        sl = pl.ds(cc * 16, 16)
                        ob[k * _SUB + r, sl] = ra[r, sl] + rb[r, sl]
                    return c2

                lax.fori_loop(0, _SUB, row, 0)
                nxt = t + 4

                @pl.when(nxt < spw)
                def _():
                    issue(nxt, s)

            pltpu.async_copy(
                ob,
                out_hbm.at[pl.ds(wid * epw + m * _GM * _SUB, _GM * _SUB)],
                sws[mm])
        return carry

    lax.fori_loop(0, gnm // 2, two_megas, 0)
    for mm in range(2):
        pltpu.make_async_copy(
            obs[mm], out_hbm.at[pl.ds(0, _GM * _SUB)], sws[mm]).wait()


@functools.cache
def _sc_scatter_add_kernel(ne):
    spw = ne // _NW // _SUB
    return pl.kernel(
        functools.partial(_sc_scatter_add_body, ne),
        out_type=jax.ShapeDtypeStruct((_NC, N, H), jnp.float32),
        mesh=_sc_mesh(),
        scratch_types=[
            pltpu.VMEM((spw, _SUB), jnp.int32),
            [pltpu.VMEM((_SM * _SUB, H), jnp.float32)] * 2,
            pltpu.VMEM_SHARED((N, H), jnp.float32),
            [pltpu.SemaphoreType.DMA] * 2,
            [pltpu.SemaphoreType.DMA] * 2,
        ],
        compiler_params=pltpu.CompilerParams(use_tc_tiling_on_sc=False),
    )


def _sc_scatter_add(he, dst2):
    ne = dst2.shape[0] * _SUB
    return _sc_scatter_add_kernel(ne)(he, dst2)


def _sc_scatter_add_body(ne, he_hbm, dst_hbm, out_hbm, didx, rs, shared,
                         lsems, ssems):
    epw = ne // _NW
    spw = epw // _SUB
    snm = spw // _SM
    cid = lax.axis_index("c")
    sid = lax.axis_index("s")
    wid = sid * _NC + cid
    base = wid * epw
    mrows = _SM * _SUB

    def zrow(r, c2):
        for cc in range(H // 16):
            rs[0][r, pl.ds(cc * 16, 16)] = jnp.zeros((16,), jnp.float32)
        return c2

    lax.fori_loop(0, _NPS, zrow, 0)
    pltpu.sync_copy(rs[0], shared.at[pl.ds(sid * _NPS, _NPS)])
    pltpu.sync_copy(dst_hbm.at[pl.ds(wid * spw, spw)], didx)
    plsc.subcore_barrier()

    def load(m, s):
        pltpu.async_copy(he_hbm.at[pl.ds(base + m * mrows, mrows)],
                         rs[s], lsems[s])

    def wait_load(s):
        pltpu.make_async_copy(he_hbm.at[pl.ds(0, mrows)], rs[s],
                              lsems[s]).wait()

    def wait_scatter(s):
        pltpu.make_async_copy(rs[s], shared.at[pl.ds(0, mrows)],
                              ssems[s]).wait()

    load(0, 0)

    def two_megas(ii, carry):
        for mm in range(2):
            m = 2 * ii + mm
            wait_load(mm)

            @pl.when(m + 1 < snm)
            def _():
                @pl.when(m >= 1)
                def _():
                    wait_scatter(1 - mm)

                load(m + 1, 1 - mm)

            for b in range(_SM):
                t = _SM * m + b
                pltpu.async_copy(rs[mm].at[pl.ds(b * _SUB, _SUB)],
                                 shared.at[didx.at[t]], ssems[mm], add=True)
        return carry

    lax.fori_loop(0, snm // 2, two_megas, 0)
    wait_scatter(0)
    wait_scatter(1)
    plsc.subcore_barrier()
    pltpu.sync_copy(shared.at[pl.ds(sid * _NPS, _NPS)],
                    out_hbm.at[cid, pl.ds(sid * _NPS, _NPS)])


def kernel(x, edge_attr, pe, edge_index,
           enc_W1, enc_b1, enc_W2, enc_b2,
           ee_W1, ee_b1, ee_W2, ee_b2,
           eu_W1, eu_b1, eu_W2, eu_b2,
           nu_W1, nu_b1, nu_W2, nu_b2,
           fu_W1, fu_b1, fu_W2, fu_b2,
           dec_W1, dec_b1, dec_W2, dec_b2):
    src2 = edge_index[0].astype(jnp.int32).reshape(E // _SUB, _SUB)
    dst2 = edge_index[1].astype(jnp.int32).reshape(E // _SUB, _SUB)
    sh = EH // _SUB
    src2a, src2b = src2[:sh], src2[sh:]
    dst2a, dst2b = dst2[:sh], dst2[sh:]

    enc_w1x = enc_W1[:, :D_IN, :]
    enc_w1p = enc_W1[:, D_IN:, :]
    enc_b1r = enc_b1.reshape(T, 1, H)
    enc_b2r = enc_b2.reshape(T, 1, H)
    eu_wa = [eu_W1[l, :H] for l in range(L)]
    eu_wb = [eu_W1[l, H:2 * H] for l in range(L)]
    eu_wc = [eu_W1[l, 2 * H:] for l in range(L)]
    eu_b1r = [eu_b1[l].reshape(1, H) for l in range(L)]
    eu_b2r = [eu_b2[l].reshape(1, H) for l in range(L)]
    nu_w1h = [nu_W1[l, :H] for l in range(L)]
    nu_w1m = [nu_W1[l, H:] for l in range(L)]

    h, A, B = _enc_call(x, pe, enc_w1x, enc_w1p, enc_b1r, enc_W2, enc_b2r,
                        eu_wa[0], eu_wb[0])
    eeb1 = ee_b1.reshape(1, H)
    eeb2 = ee_b2.reshape(1, H)
    hea = _eenc_call(edge_attr, 0, ee_W1, eeb1, ee_W2, eeb2)
    heb = _eenc_call(edge_attr, EH // RE, ee_W1, eeb1, ee_W2, eeb2)

    for l in range(L):
        Ga = _sc_gather_add(A, B, src2a, dst2a)
        Gb = _sc_gather_add(A, B, src2b, dst2b)
        hea = _eupd_call(Ga, hea, eu_wc[l], eu_b1r[l], eu_W2[l], eu_b2r[l])
        ma = _sc_scatter_add(hea, dst2a)
        heb = _eupd_call(Gb, heb, eu_wc[l], eu_b1r[l], eu_W2[l], eu_b2r[l])
        mb = _sc_scatter_add(heb, dst2b)
        args = (h, ma, mb, nu_w1h[l], nu_w1m[l],
                nu_b1[l].reshape(1, H), nu_W2[l], nu_b2[l].reshape(1, H),
                fu_W1[l], fu_b1[l].reshape(1, H), fu_W2[l],
                fu_b2[l].reshape(1, H))
        if l + 1 < L:
            h, A, B = _nupd_call(*args, eu_wa[l + 1], eu_wb[l + 1])
        else:
            h = _nupd_call(*args)

    return _dec_call(x, h, dec_W1, dec_b1.reshape(T, 1, H), dec_W2,
                     dec_b2.reshape(T, 1, OUT))


# speedup vs baseline: 5.3198x; 1.6361x over previous
import functools

import jax
import jax.numpy as jnp
from jax import lax
from jax.experimental import pallas as pl
from jax.experimental.pallas import tpu as pltpu
from jax.experimental.pallas import tpu_sc as plsc

N = 10000
E = 320000
D_IN = 128
PE = 16
H = 64
OUT = 128
T = 3
L = 3

RN = 2000
RE = 4000

_NC = 2
_NS = 16
_NW = _NC * _NS
_SUB = 125
_GM = 2
_SM = 5
_NPS = N // _NS
EH = E // 2

@functools.cache
def _sc_mesh():
    return plsc.VectorSubcoreMesh(core_axis_name="c", subcore_axis_name="s",
                                  num_cores=_NC, num_subcores=_NS)


def _relu(v):
    return jnp.maximum(v, 0.0)


def _dot(a, b):
    return jnp.dot(a, b, preferred_element_type=jnp.float32)


def _full(a):
    return pl.BlockSpec(a.shape, lambda i: (0,) * a.ndim)


def _rows(shape):
    nd = len(shape)
    return pl.BlockSpec(shape, lambda i: (i,) + (0,) * (nd - 1))


def _enc_body(x_ref, pe_ref, w1x, w1p, b1, w2, b2, wa, wb,
              h_ref, a_ref, b_ref):
    xb = x_ref[...]
    peb = pe_ref[...]
    nt = jnp.clip(jnp.abs(xb[:, 2:3]).astype(jnp.int32), 0, T - 1)
    h = jnp.zeros((RN, H), jnp.float32)
    for t in range(T):
        h1 = _relu(_dot(xb, w1x[t]) + _dot(peb, w1p[t]) + b1[t])
        ht = _dot(h1, w2[t]) + b2[t]
        h = jnp.where(nt == t, ht, h)
    h_ref[...] = h
    a_ref[...] = _dot(h, wa[...])
    b_ref[...] = _dot(h, wb[...])


def _enc_call(x, pe, w1x, w1p, b1, w2, b2, wa, wb):
    return pl.pallas_call(
        _enc_body,
        grid=(N // RN,),
        in_specs=[_rows((RN, D_IN)), _rows((RN, PE))]
        + [_full(w) for w in (w1x, w1p, b1, w2, b2, wa, wb)],
        out_specs=[_rows((RN, H))] * 3,
        out_shape=[jax.ShapeDtypeStruct((N, H), jnp.float32)] * 3,
    )(x, pe, w1x, w1p, b1, w2, b2, wa, wb)


def _eenc_body(ea_ref, w1, b1, w2, b2, he_ref):
    h1 = _relu(_dot(ea_ref[...], w1[...]) + b1[...])
    he_ref[...] = _dot(h1, w2[...]) + b2[...]


def _eenc_call(ea, off, w1, b1, w2, b2):
    return pl.pallas_call(
        _eenc_body,
        grid=(EH // RE,),
        in_specs=[pl.BlockSpec((RE, PE), lambda i: (i + off, 0))]
        + [_full(w) for w in (w1, b1, w2, b2)],
        out_specs=_rows((RE, H)),
        out_shape=jax.ShapeDtypeStruct((EH, H), jnp.float32),
    )(ea, w1, b1, w2, b2)


def _eupd_body(g_ref, he_ref, wc, bc, w2, b2, out_he):
    he = he_ref[...]
    z = _relu(g_ref[...] + _dot(he, wc[...]) + bc[...])
    out_he[...] = he + _dot(z, w2[...]) + b2[...]


def _eupd_call(g, he, wc2, bc2, w22, b22):
    nr = g.shape[0]
    rb = RE // 2
    return pl.pallas_call(
        _eupd_body,
        grid=(nr // rb,),
        in_specs=[_rows((rb, 2 * H))] * 2
        + [_full(w) for w in (wc2, bc2, w22, b22)],
        out_specs=_rows((rb, 2 * H)),
        out_shape=jax.ShapeDtypeStruct((nr, 2 * H), jnp.float32),
    )(g, he, wc2, bc2, w22, b22)


def _nupd_body(h_ref, ma_ref, mb_ref, w1h, w1m, b1, w2, b2,
               f1, fb1, f2, fb2, wa, wb, h_out, a_out, b_out):
    h = h_ref[...]
    m = ma_ref[0] + ma_ref[1] + mb_ref[0] + mb_ref[1]
    u = _dot(_relu(_dot(h, w1h[...]) + _dot(m, w1m[...]) + b1[...]), w2[...]) + b2[...]
    local = h + u
    hn = _dot(_relu(_dot(local, f1[...]) + fb1[...]), f2[...]) + fb2[...]
    h_out[...] = hn
    a_out[...] = _dot(hn, wa[...])
    b_out[...] = _dot(hn, wb[...])


def _nupd_last_body(h_ref, ma_ref, mb_ref, w1h, w1m, b1, w2, b2,
                    f1, fb1, f2, fb2, h_out):
    h = h_ref[...]
    m = ma_ref[0] + ma_ref[1] + mb_ref[0] + mb_ref[1]
    u = _dot(_relu(_dot(h, w1h[...]) + _dot(m, w1m[...]) + b1[...]), w2[...]) + b2[...]
    local = h + u
    h_out[...] = _dot(_relu(_dot(local, f1[...]) + fb1[...]), f2[...]) + fb2[...]


_mspec = pl.BlockSpec((_NC, RN, H), lambda i: (0, i, 0))


def _nupd_call(h, ma, mb, w1h, w1m, b1, w2, b2, f1, fb1, f2, fb2,
               wa=None, wb=None):
    if wa is None:
        return pl.pallas_call(
            _nupd_last_body,
            grid=(N // RN,),
            in_specs=[_rows((RN, H)), _mspec, _mspec]
            + [_full(w) for w in (w1h, w1m, b1, w2, b2, f1, fb1, f2, fb2)],
            out_specs=[_rows((RN, H))],
            out_shape=[jax.ShapeDtypeStruct((N, H), jnp.float32)],
        )(h, ma, mb, w1h, w1m, b1, w2, b2, f1, fb1, f2, fb2)[0]
    return pl.pallas_call(
        _nupd_body,
        grid=(N // RN,),
        in_specs=[_rows((RN, H)), _mspec, _mspec]
        + [_full(w) for w in (w1h, w1m, b1, w2, b2, f1, fb1, f2, fb2, wa, wb)],
        out_specs=[_rows((RN, H))] * 3,
        out_shape=[jax.ShapeDtypeStruct((N, H), jnp.float32)] * 3,
    )(h, ma, mb, w1h, w1m, b1, w2, b2, f1, fb1, f2, fb2, wa, wb)


def _dec_body(x_ref, h_ref, w1, b1, w2, b2, out_ref):
    nt = jnp.clip(jnp.abs(x_ref[:, 2:3]).astype(jnp.int32), 0, T - 1)
    h = h_ref[...]
    out = jnp.zeros((RN, OUT), jnp.float32)
    for t in range(T):
        ot = _dot(_relu(_dot(h, w1[t]) + b1[t]), w2[t]) + b2[t]
        out = jnp.where(nt == t, ot, out)
    out_ref[...] = out


def _dec_call(x, h, w1, b1, w2, b2):
    return pl.pallas_call(
        _dec_body,
        grid=(N // RN,),
        in_specs=[_rows((RN, D_IN)), _rows((RN, H))]
        + [_full(w) for w in (w1, b1, w2, b2)],
        out_specs=_rows((RN, OUT)),
        out_shape=jax.ShapeDtypeStruct((N, OUT), jnp.float32),
    )(x, h, w1, b1, w2, b2)


@functools.cache
def _sc_gather_add_kernel(ne):
    spw = ne // _NW // _SUB
    return pl.kernel(
        functools.partial(_sc_gather_add_body, ne),
        out_type=jax.ShapeDtypeStruct((ne, H), jnp.float32),
        mesh=_sc_mesh(),
        scratch_types=[
            pltpu.VMEM((spw, _SUB), jnp.int32),
            pltpu.VMEM((spw, _SUB), jnp.int32),
            [pltpu.VMEM((_SUB, H), jnp.float32)] * 4,
            [pltpu.VMEM((_SUB, H), jnp.float32)] * 4,
            [pltpu.VMEM((_GM * _SUB, H), jnp.float32)] * 2,
            [pltpu.SemaphoreType.DMA] * 4,
            [pltpu.SemaphoreType.DMA] * 2,
        ],
        compiler_params=pltpu.CompilerParams(use_tc_tiling_on_sc=False),
    )


def _sc_gather_add(a, b, src2, dst2):
    ne = src2.shape[0] * _SUB
    return _sc_gather_add_kernel(ne)(a, b, src2, dst2)


def _sc_gather_add_body(ne, a_hbm, b_hbm, src_hbm, dst_hbm, out_hbm,
                        sidx, didx, ras, rbs, obs, sgs, sws):
    epw = ne // _NW
    spw = epw // _SUB
    gnm = spw // _GM
    wid = lax.axis_index("s") * _NC + lax.axis_index("c")
    pltpu.sync_copy(src_hbm.at[pl.ds(wid * spw, spw)], sidx)
    pltpu.sync_copy(dst_hbm.at[pl.ds(wid * spw, spw)], didx)

    def issue(t, s):
        pltpu.async_copy(a_hbm.at[sidx.at[t]], ras[s], sgs[s])
        pltpu.async_copy(b_hbm.at[didx.at[t]], rbs[s], sgs[s])

    def wait_gather(s):
        pltpu.make_async_copy(a_hbm.at[sidx.at[0]], ras[s], sgs[s]).wait()
        pltpu.make_async_copy(b_hbm.at[didx.at[0]], rbs[s], sgs[s]).wait()

    for s in range(4):
        issue(s, s)

    def two_megas(ii, carry):
        for mm in range(2):
            m = 2 * ii + mm
            ob = obs[mm]
            @pl.when(m >= 2)
            def _():
                pltpu.make_async_copy(
                    ob, out_hbm.at[pl.ds(0, _GM * _SUB)], sws[mm]).wait()

            for k in range(_GM):
                s = 2 * mm + k
                t = _GM * m + k
                wait_gather(s)
                ra, rb = ras[s], rbs[s]

                def row(r, c2, ra=ra, rb=rb, ob=ob, k=k):
                    for cc in range(H // 16):
                        sl = pl.ds(cc * 16, 16)
                        ob[k * _SUB + r, sl] = ra[r, sl] + rb[r, sl]
                    return c2

                lax.fori_loop(0, _SUB, row, 0)
                nxt = t + 4

                @pl.when(nxt < spw)
                def _():
                    issue(nxt, s)

            pltpu.async_copy(
                ob,
                out_hbm.at[pl.ds(wid * epw + m * _GM * _SUB, _GM * _SUB)],
                sws[mm])
        return carry

    lax.fori_loop(0, gnm // 2, two_megas, 0)
    for mm in range(2):
        pltpu.make_async_copy(
            obs[mm], out_hbm.at[pl.ds(0, _GM * _SUB)], sws[mm]).wait()


@functools.cache
def _sc_scatter_add_kernel(ne):
    spw = ne // _NW // _SUB
    return pl.kernel(
        functools.partial(_sc_scatter_add_body, ne),
        out_type=jax.ShapeDtypeStruct((_NC, N, H), jnp.float32),
        mesh=_sc_mesh(),
        scratch_types=[
            pltpu.VMEM((spw, _SUB), jnp.int32),
            [pltpu.VMEM((_SM * _SUB, H), jnp.float32)] * 2,
            pltpu.VMEM_SHARED((N, H), jnp.float32),
            [pltpu.SemaphoreType.DMA] * 2,
            [pltpu.SemaphoreType.DMA] * 2,
        ],
        compiler_params=pltpu.CompilerParams(use_tc_tiling_on_sc=False),
    )


def _sc_scatter_add(he, dst2):
    ne = dst2.shape[0] * _SUB
    return _sc_scatter_add_kernel(ne)(he, dst2)


def _sc_scatter_add_body(ne, he_hbm, dst_hbm, out_hbm, didx, rs, shared,
                         lsems, ssems):
    epw = ne // _NW
    spw = epw // _SUB
    snm = spw // _SM
    cid = lax.axis_index("c")
    sid = lax.axis_index("s")
    wid = sid * _NC + cid
    base = wid * epw
    mrows = _SM * _SUB

    def zrow(r, c2):
        for cc in range(H // 16):
            rs[0][r, pl.ds(cc * 16, 16)] = jnp.zeros((16,), jnp.float32)
        return c2

    lax.fori_loop(0, _NPS, zrow, 0)
    pltpu.sync_copy(rs[0], shared.at[pl.ds(sid * _NPS, _NPS)])
    pltpu.sync_copy(dst_hbm.at[pl.ds(wid * spw, spw)], didx)
    plsc.subcore_barrier()

    def load(m, s):
        pltpu.async_copy(he_hbm.at[pl.ds(base + m * mrows, mrows)],
                         rs[s], lsems[s])

    def wait_load(s):
        pltpu.make_async_copy(he_hbm.at[pl.ds(0, mrows)], rs[s],
                              lsems[s]).wait()

    def wait_scatter(s):
        pltpu.make_async_copy(rs[s], shared.at[pl.ds(0, mrows)],
                              ssems[s]).wait()

    load(0, 0)

    def two_megas(ii, carry):
        for mm in range(2):
            m = 2 * ii + mm
            wait_load(mm)

            @pl.when(m + 1 < snm)
            def _():
                @pl.when(m >= 1)
                def _():
                    wait_scatter(1 - mm)

                load(m + 1, 1 - mm)

            for b in range(_SM):
                t = _SM * m + b
                pltpu.async_copy(rs[mm].at[pl.ds(b * _SUB, _SUB)],
                                 shared.at[didx.at[t]], ssems[mm], add=True)
        return carry

    lax.fori_loop(0, snm // 2, two_megas, 0)
    wait_scatter(0)
    wait_scatter(1)
    plsc.subcore_barrier()
    pltpu.sync_copy(shared.at[pl.ds(sid * _NPS, _NPS)],
                    out_hbm.at[cid, pl.ds(sid * _NPS, _NPS)])


def kernel(x, edge_attr, pe, edge_index,
           enc_W1, enc_b1, enc_W2, enc_b2,
           ee_W1, ee_b1, ee_W2, ee_b2,
           eu_W1, eu_b1, eu_W2, eu_b2,
           nu_W1, nu_b1, nu_W2, nu_b2,
           fu_W1, fu_b1, fu_W2, fu_b2,
           dec_W1, dec_b1, dec_W2, dec_b2):
    src2 = edge_index[0].astype(jnp.int32).reshape(E // _SUB, _SUB)
    dst2 = edge_index[1].astype(jnp.int32).reshape(E // _SUB, _SUB)
    sh = EH // _SUB
    src2a, src2b = src2[:sh], src2[sh:]
    dst2a, dst2b = dst2[:sh], dst2[sh:]

    enc_w1x = enc_W1[:, :D_IN, :]
    enc_w1p = enc_W1[:, D_IN:, :]
    enc_b1r = enc_b1.reshape(T, 1, H)
    enc_b2r = enc_b2.reshape(T, 1, H)
    eu_wa = [eu_W1[l, :H] for l in range(L)]
    eu_wb = [eu_W1[l, H:2 * H] for l in range(L)]

    def bd(w):
        return jnp.zeros((2 * H, 2 * H), jnp.float32).at[:H, :H].set(
            w).at[H:, H:].set(w)

    def b2x(b):
        return jnp.tile(b.reshape(1, H), (1, 2))

    eu_wc2 = [bd(eu_W1[l, 2 * H:]) for l in range(L)]
    eu_w222 = [bd(eu_W2[l]) for l in range(L)]
    eu_b1r2 = [b2x(eu_b1[l]) for l in range(L)]
    eu_b2r2 = [b2x(eu_b2[l]) for l in range(L)]
    nu_w1h = [nu_W1[l, :H] for l in range(L)]
    nu_w1m = [nu_W1[l, H:] for l in range(L)]

    h, A, B = _enc_call(x, pe, enc_w1x, enc_w1p, enc_b1r, enc_W2, enc_b2r,
                        eu_wa[0], eu_wb[0])
    eeb1 = ee_b1.reshape(1, H)
    eeb2 = ee_b2.reshape(1, H)
    pk = (EH // 2, 2 * H)
    hea = _eenc_call(edge_attr, 0, ee_W1, eeb1, ee_W2, eeb2).reshape(pk)
    heb = _eenc_call(edge_attr, EH // RE, ee_W1, eeb1, ee_W2,
                     eeb2).reshape(pk)

    for l in range(L):
        Ga = _sc_gather_add(A, B, src2a, dst2a)
        Gb = _sc_gather_add(A, B, src2b, dst2b)
        hea = _eupd_call(Ga.reshape(pk), hea, eu_wc2[l], eu_b1r2[l],
                         eu_w222[l], eu_b2r2[l])
        ma = _sc_scatter_add(hea.reshape(EH, H), dst2a)
        heb = _eupd_call(Gb.reshape(pk), heb, eu_wc2[l], eu_b1r2[l],
                         eu_w222[l], eu_b2r2[l])
        mb = _sc_scatter_add(heb.reshape(EH, H), dst2b)
        args = (h, ma, mb, nu_w1h[l], nu_w1m[l],
                nu_b1[l].reshape(1, H), nu_W2[l], nu_b2[l].reshape(1, H),
                fu_W1[l], fu_b1[l].reshape(1, H), fu_W2[l],
                fu_b2[l].reshape(1, H))
        if l + 1 < L:
            h, A, B = _nupd_call(*args, eu_wa[l + 1], eu_wb[l + 1])
        else:
            h = _nupd_call(*args)

    return _dec_call(x, h, dec_W1, dec_b1.reshape(T, 1, H), dec_W2,
                     dec_b2.reshape(T, 1, OUT))


# speedup vs baseline: 5.8189x; 1.0938x over previous
import functools

import jax
import jax.numpy as jnp
from jax import lax
from jax.experimental import pallas as pl
from jax.experimental.pallas import tpu as pltpu
from jax.experimental.pallas import tpu_sc as plsc

N = 10000
E = 320000
D_IN = 128
PE = 16
H = 64
OUT = 128
T = 3
L = 3

RN = 2000
RE = 4000

_NC = 2
_NS = 16
_NW = _NC * _NS
_SUB = 125
_GM = 2
_SM = 5
_NPS = N // _NS
EH = E // 2

@functools.cache
def _sc_mesh():
    return plsc.VectorSubcoreMesh(core_axis_name="c", subcore_axis_name="s",
                                  num_cores=_NC, num_subcores=_NS)


def _relu(v):
    return jnp.maximum(v, 0.0)


def _dot(a, b):
    return jnp.dot(a, b, preferred_element_type=jnp.float32)


def _full(a):
    return pl.BlockSpec(a.shape, lambda i: (0,) * a.ndim)


def _rows(shape):
    nd = len(shape)
    return pl.BlockSpec(shape, lambda i: (i,) + (0,) * (nd - 1))


def _enc_body(x_ref, pe_ref, w1x, w1p, b1, w2, b2, wa, wb,
              h_ref, a_ref, b_ref):
    xb = x_ref[...]
    peb = pe_ref[...]
    nt = jnp.clip(jnp.abs(xb[:, 2:3]).astype(jnp.int32), 0, T - 1)
    h = jnp.zeros((RN, H), jnp.float32)
    for t in range(T):
        h1 = _relu(_dot(xb, w1x[t]) + _dot(peb, w1p[t]) + b1[t])
        ht = _dot(h1, w2[t]) + b2[t]
        h = jnp.where(nt == t, ht, h)
    h_ref[...] = h
    a_ref[...] = _dot(h, wa[...])
    b_ref[...] = _dot(h, wb[...])


def _enc_call(x, pe, w1x, w1p, b1, w2, b2, wa, wb):
    return pl.pallas_call(
        _enc_body,
        grid=(N // RN,),
        in_specs=[_rows((RN, D_IN)), _rows((RN, PE))]
        + [_full(w) for w in (w1x, w1p, b1, w2, b2, wa, wb)],
        out_specs=[_rows((RN, H))] * 3,
        out_shape=[jax.ShapeDtypeStruct((N, H), jnp.float32)] * 3,
    )(x, pe, w1x, w1p, b1, w2, b2, wa, wb)


def _eenc_body(ea1_ref, ea2_ref, w1t, b1c, w2t, b2c, out_ref):
    h1 = _relu(_dot(w1t[...], ea1_ref[...]) + b1c[...])
    he1 = _dot(w2t[...], h1) + b2c[...]
    h2 = _relu(_dot(w1t[...], ea2_ref[...]) + b1c[...])
    he2 = _dot(w2t[...], h2) + b2c[...]
    out_ref[:, :H] = he1.T
    out_ref[:, H:] = he2.T


def _eenc_call(eat, half, w1t, b1c, w2t, b2c):
    rb = 3200
    nb = (EH // 2) // rb
    off = half * 2 * nb
    return pl.pallas_call(
        _eenc_body,
        grid=(nb,),
        in_specs=[pl.BlockSpec((PE, rb), lambda i: (0, i + off)),
                  pl.BlockSpec((PE, rb), lambda i: (0, i + off + nb))]
        + [_full(w) for w in (w1t, b1c, w2t, b2c)],
        out_specs=_rows((rb, 2 * H)),
        out_shape=jax.ShapeDtypeStruct((EH // 2, 2 * H), jnp.float32),
    )(eat, eat, w1t, b1c, w2t, b2c)


def _eupd_body(g_ref, he_ref, wc, bc, w2, b2, out_he):
    he = he_ref[...]
    z = _relu(g_ref[...] + _dot(he, wc[...]) + bc[...])
    out_he[...] = he + _dot(z, w2[...]) + b2[...]


def _eupd_call(g, he, wc2, bc2, w22, b22):
    nr = g.shape[0]
    rb = RE // 2
    return pl.pallas_call(
        _eupd_body,
        grid=(nr // rb,),
        in_specs=[_rows((rb, 2 * H))] * 2
        + [_full(w) for w in (wc2, bc2, w22, b22)],
        out_specs=_rows((rb, 2 * H)),
        out_shape=jax.ShapeDtypeStruct((nr, 2 * H), jnp.float32),
    )(g, he, wc2, bc2, w22, b22)


def _nupd_body(h_ref, ma_ref, mb_ref, w1h, w1m, b1, w2, b2,
               f1, fb1, f2, fb2, wa, wb, h_out, a_out, b_out):
    h = h_ref[...]
    m = ma_ref[0] + ma_ref[1] + mb_ref[0] + mb_ref[1]
    u = _dot(_relu(_dot(h, w1h[...]) + _dot(m, w1m[...]) + b1[...]), w2[...]) + b2[...]
    local = h + u
    hn = _dot(_relu(_dot(local, f1[...]) + fb1[...]), f2[...]) + fb2[...]
    h_out[...] = hn
    a_out[...] = _dot(hn, wa[...])
    b_out[...] = _dot(hn, wb[...])


def _nupd_last_body(h_ref, ma_ref, mb_ref, w1h, w1m, b1, w2, b2,
                    f1, fb1, f2, fb2, h_out):
    h = h_ref[...]
    m = ma_ref[0] + ma_ref[1] + mb_ref[0] + mb_ref[1]
    u = _dot(_relu(_dot(h, w1h[...]) + _dot(m, w1m[...]) + b1[...]), w2[...]) + b2[...]
    local = h + u
    h_out[...] = _dot(_relu(_dot(local, f1[...]) + fb1[...]), f2[...]) + fb2[...]


_mspec = pl.BlockSpec((_NC, RN, H), lambda i: (0, i, 0))


def _nupd_call(h, ma, mb, w1h, w1m, b1, w2, b2, f1, fb1, f2, fb2,
               wa=None, wb=None):
    if wa is None:
        return pl.pallas_call(
            _nupd_last_body,
            grid=(N // RN,),
            in_specs=[_rows((RN, H)), _mspec, _mspec]
            + [_full(w) for w in (w1h, w1m, b1, w2, b2, f1, fb1, f2, fb2)],
            out_specs=[_rows((RN, H))],
            out_shape=[jax.ShapeDtypeStruct((N, H), jnp.float32)],
        )(h, ma, mb, w1h, w1m, b1, w2, b2, f1, fb1, f2, fb2)[0]
    return pl.pallas_call(
        _nupd_body,
        grid=(N // RN,),
        in_specs=[_rows((RN, H)), _mspec, _mspec]
        + [_full(w) for w in (w1h, w1m, b1, w2, b2, f1, fb1, f2, fb2, wa, wb)],
        out_specs=[_rows((RN, H))] * 3,
        out_shape=[jax.ShapeDtypeStruct((N, H), jnp.float32)] * 3,
    )(h, ma, mb, w1h, w1m, b1, w2, b2, f1, fb1, f2, fb2, wa, wb)


def _dec_body(x_ref, h_ref, w1, b1, w2, b2, out_ref):
    nt = jnp.clip(jnp.abs(x_ref[:, 2:3]).astype(jnp.int32), 0, T - 1)
    h = h_ref[...]
    out = jnp.zeros((RN, OUT), jnp.float32)
    for t in range(T):
        ot = _dot(_relu(_dot(h, w1[t]) + b1[t]), w2[t]) + b2[t]
        out = jnp.where(nt == t, ot, out)
    out_ref[...] = out


def _dec_call(x, h, w1, b1, w2, b2):
    return pl.pallas_call(
        _dec_body,
        grid=(N // RN,),
        in_specs=[_rows((RN, D_IN)), _rows((RN, H))]
        + [_full(w) for w in (w1, b1, w2, b2)],
        out_specs=_rows((RN, OUT)),
        out_shape=jax.ShapeDtypeStruct((N, OUT), jnp.float32),
    )(x, h, w1, b1, w2, b2)


@functools.cache
def _sc_gather_add_kernel(ne):
    spw = ne // _NW // _SUB
    return pl.kernel(
        functools.partial(_sc_gather_add_body, ne),
        out_type=jax.ShapeDtypeStruct((ne, H), jnp.float32),
        mesh=_sc_mesh(),
        scratch_types=[
            pltpu.VMEM((spw, _SUB), jnp.int32),
            pltpu.VMEM((spw, _SUB), jnp.int32),
            [pltpu.VMEM((_SUB, H), jnp.float32)] * 4,
            [pltpu.VMEM((_SUB, H), jnp.float32)] * 4,
            [pltpu.VMEM((_GM * _SUB, H), jnp.float32)] * 2,
            [pltpu.SemaphoreType.DMA] * 4,
            [pltpu.SemaphoreType.DMA] * 2,
        ],
        compiler_params=pltpu.CompilerParams(use_tc_tiling_on_sc=False),
    )


def _sc_gather_add(a, b, src2, dst2):
    ne = src2.shape[0] * _SUB
    return _sc_gather_add_kernel(ne)(a, b, src2, dst2)


def _sc_gather_add_body(ne, a_hbm, b_hbm, src_hbm, dst_hbm, out_hbm,
                        sidx, didx, ras, rbs, obs, sgs, sws):
    epw = ne // _NW
    spw = epw // _SUB
    gnm = spw // _GM
    wid = lax.axis_index("s") * _NC + lax.axis_index("c")
    pltpu.sync_copy(src_hbm.at[pl.ds(wid * spw, spw)], sidx)
    pltpu.sync_copy(dst_hbm.at[pl.ds(wid * spw, spw)], didx)

    def issue(t, s):
        pltpu.async_copy(a_hbm.at[sidx.at[t]], ras[s], sgs[s])
        pltpu.async_copy(b_hbm.at[didx.at[t]], rbs[s], sgs[s])

    def wait_gather(s):
        pltpu.make_async_copy(a_hbm.at[sidx.at[0]], ras[s], sgs[s]).wait()
        pltpu.make_async_copy(b_hbm.at[didx.at[0]], rbs[s], sgs[s]).wait()

    for s in range(4):
        issue(s, s)

    def two_megas(ii, carry):
        for mm in range(2):
            m = 2 * ii + mm
            ob = obs[mm]
            @pl.when(m >= 2)
            def _():
                pltpu.make_async_copy(
                    ob, out_hbm.at[pl.ds(0, _GM * _SUB)], sws[mm]).wait()

            for k in range(_GM):
                s = 2 * mm + k
                t = _GM * m + k
                wait_gather(s)
                ra, rb = ras[s], rbs[s]

                def row(r, c2, ra=ra, rb=rb, ob=ob, k=k):
                    for cc in range(H // 16):
                        sl = pl.ds(cc * 16, 16)
                        ob[k * _SUB + r, sl] = ra[r, sl] + rb[r, sl]
                    return c2

                lax.fori_loop(0, _SUB, row, 0)
                nxt = t + 4

                @pl.when(nxt < spw)
                def _():
                    issue(nxt, s)

            pltpu.async_copy(
                ob,
                out_hbm.at[pl.ds(wid * epw + m * _GM * _SUB, _GM * _SUB)],
                sws[mm])
        return carry

    lax.fori_loop(0, gnm // 2, two_megas, 0)
    for mm in range(2):
        pltpu.make_async_copy(
            obs[mm], out_hbm.at[pl.ds(0, _GM * _SUB)], sws[mm]).wait()


@functools.cache
def _sc_scatter_add_kernel(ne):
    spw = ne // _NW // _SUB
    return pl.kernel(
        functools.partial(_sc_scatter_add_body, ne),
        out_type=jax.ShapeDtypeStruct((_NC, N, H), jnp.float32),
        mesh=_sc_mesh(),
        scratch_types=[
            pltpu.VMEM((spw, _SUB), jnp.int32),
            [pltpu.VMEM((_SM * _SUB, H), jnp.float32)] * 2,
            pltpu.VMEM_SHARED((N, H), jnp.float32),
            [pltpu.SemaphoreType.DMA] * 2,
            [pltpu.SemaphoreType.DMA] * 2,
        ],
        compiler_params=pltpu.CompilerParams(use_tc_tiling_on_sc=False),
    )


def _sc_scatter_add(he, dst2):
    ne = dst2.shape[0] * _SUB
    return _sc_scatter_add_kernel(ne)(he, dst2)


def _sc_scatter_add_body(ne, he_hbm, dst_hbm, out_hbm, didx, rs, shared,
                         lsems, ssems):
    epw = ne // _NW
    spw = epw // _SUB
    snm = spw // _SM
    cid = lax.axis_index("c")
    sid = lax.axis_index("s")
    wid = sid * _NC + cid
    base = wid * epw
    mrows = _SM * _SUB

    def zrow(r, c2):
        for cc in range(H // 16):
            rs[0][r, pl.ds(cc * 16, 16)] = jnp.zeros((16,), jnp.float32)
        return c2

    lax.fori_loop(0, _NPS, zrow, 0)
    pltpu.sync_copy(rs[0], shared.at[pl.ds(sid * _NPS, _NPS)])
    pltpu.sync_copy(dst_hbm.at[pl.ds(wid * spw, spw)], didx)
    plsc.subcore_barrier()

    def load(m, s):
        pltpu.async_copy(he_hbm.at[pl.ds(base + m * mrows, mrows)],
                         rs[s], lsems[s])

    def wait_load(s):
        pltpu.make_async_copy(he_hbm.at[pl.ds(0, mrows)], rs[s],
                              lsems[s]).wait()

    def wait_scatter(s):
        pltpu.make_async_copy(rs[s], shared.at[pl.ds(0, mrows)],
                              ssems[s]).wait()

    load(0, 0)

    def two_megas(ii, carry):
        for mm in range(2):
            m = 2 * ii + mm
            wait_load(mm)

            @pl.when(m + 1 < snm)
            def _():
                @pl.when(m >= 1)
                def _():
                    wait_scatter(1 - mm)

                load(m + 1, 1 - mm)

            for b in range(_SM):
                t = _SM * m + b
                pltpu.async_copy(rs[mm].at[pl.ds(b * _SUB, _SUB)],
                                 shared.at[didx.at[t]], ssems[mm], add=True)
        return carry

    lax.fori_loop(0, snm // 2, two_megas, 0)
    wait_scatter(0)
    wait_scatter(1)
    plsc.subcore_barrier()
    pltpu.sync_copy(shared.at[pl.ds(sid * _NPS, _NPS)],
                    out_hbm.at[cid, pl.ds(sid * _NPS, _NPS)])


def kernel(x, edge_attr, pe, edge_index,
           enc_W1, enc_b1, enc_W2, enc_b2,
           ee_W1, ee_b1, ee_W2, ee_b2,
           eu_W1, eu_b1, eu_W2, eu_b2,
           nu_W1, nu_b1, nu_W2, nu_b2,
           fu_W1, fu_b1, fu_W2, fu_b2,
           dec_W1, dec_b1, dec_W2, dec_b2):
    def ilv(v):
        return jnp.stack([v[:EH // 2], v[EH // 2:]],
                         axis=1).reshape(EH // _SUB, _SUB)

    src = edge_index[0].astype(jnp.int32)
    dst = edge_index[1].astype(jnp.int32)
    src2a, src2b = ilv(src[:EH]), ilv(src[EH:])
    dst2a, dst2b = ilv(dst[:EH]), ilv(dst[EH:])

    enc_w1x = enc_W1[:, :D_IN, :]
    enc_w1p = enc_W1[:, D_IN:, :]
    enc_b1r = enc_b1.reshape(T, 1, H)
    enc_b2r = enc_b2.reshape(T, 1, H)
    eu_wa = [eu_W1[l, :H] for l in range(L)]
    eu_wb = [eu_W1[l, H:2 * H] for l in range(L)]

    def bd(w):
        return jnp.zeros((2 * H, 2 * H), jnp.float32).at[:H, :H].set(
            w).at[H:, H:].set(w)

    def b2x(b):
        return jnp.tile(b.reshape(1, H), (1, 2))

    eu_wc2 = [bd(eu_W1[l, 2 * H:]) for l in range(L)]
    eu_w222 = [bd(eu_W2[l]) for l in range(L)]
    eu_b1r2 = [b2x(eu_b1[l]) for l in range(L)]
    eu_b2r2 = [b2x(eu_b2[l]) for l in range(L)]
    nu_w1h = [nu_W1[l, :H] for l in range(L)]
    nu_w1m = [nu_W1[l, H:] for l in range(L)]

    h, A, B = _enc_call(x, pe, enc_w1x, enc_w1p, enc_b1r, enc_W2, enc_b2r,
                        eu_wa[0], eu_wb[0])
    pk = (EH // 2, 2 * H)
    eat = edge_attr.T
    w1t = ee_W1.T
    w2t = ee_W2.T
    b1c = ee_b1.reshape(H, 1)
    b2c = ee_b2.reshape(H, 1)
    hea = _eenc_call(eat, 0, w1t, b1c, w2t, b2c)
    heb = _eenc_call(eat, 1, w1t, b1c, w2t, b2c)

    for l in range(L):
        Ga = _sc_gather_add(A, B, src2a, dst2a)
        Gb = _sc_gather_add(A, B, src2b, dst2b)
        hea = _eupd_call(Ga.reshape(pk), hea, eu_wc2[l], eu_b1r2[l],
                         eu_w222[l], eu_b2r2[l])
        ma = _sc_scatter_add(hea.reshape(EH, H), dst2a)
        heb = _eupd_call(Gb.reshape(pk), heb, eu_wc2[l], eu_b1r2[l],
                         eu_w222[l], eu_b2r2[l])
        mb = _sc_scatter_add(heb.reshape(EH, H), dst2b)
        args = (h, ma, mb, nu_w1h[l], nu_w1m[l],
                nu_b1[l].reshape(1, H), nu_W2[l], nu_b2[l].reshape(1, H),
                fu_W1[l], fu_b1[l].reshape(1, H), fu_W2[l],
                fu_b2[l].reshape(1, H))
        if l + 1 < L:
            h, A, B = _nupd_call(*args, eu_wa[l + 1], eu_wb[l + 1])
        else:
            h = _nupd_call(*args)

    return _dec_call(x, h, dec_W1, dec_b1.reshape(T, 1, H), dec_W2,
                     dec_b2.reshape(T, 1, OUT))
